# baseline (speedup 1.0000x reference)
import functools

import jax
import jax.numpy as jnp
from jax import lax
from jax.experimental import pallas as pl
from jax.experimental.pallas import tpu as pltpu

NUM_CODES = 8192
LATENT = 256
BETA = 0.25

CHUNK = 2736
CHUNK_PAD = 2816
N_PAD = 3 * CHUNK_PAD
PAD_SENTINEL = 1e30

BM = 512
BN = 1408
_N_BLOCKS = N_PAD // BN
_BIG_IDX = 2 ** 30


def _bf16r(x):
    return x.astype(jnp.bfloat16).astype(jnp.float32)


def _argmin_body(zsq_ref, csq_ref, z_ref, c_ref, idx_ref, minv_ref,
                 m0, m1, m2, a0, a1, a2):
    n = pl.program_id(1)
    zb = z_ref[...]
    cb = c_ref[...]
    mm = lax.dot_general(zb.astype(jnp.bfloat16), cb.astype(jnp.bfloat16),
                         (((1,), (1,)), ((), ())),
                         preferred_element_type=jnp.float32)
    dist = (zsq_ref[...] + csq_ref[...]) - mm * 2.0
    chunk = n // 2
    col0 = n * BN - chunk * (CHUNK_PAD - CHUNK)
    jidx = lax.broadcasted_iota(jnp.int32, (BM, BN), 1) + col0
    bmin = jnp.min(dist, axis=1, keepdims=True)
    bidx = jnp.min(jnp.where(dist == bmin, jidx, _BIG_IDX),
                   axis=1, keepdims=True)

    @pl.when(n == 0)
    def _init():
        for mr, ar in ((m0, a0), (m1, a1), (m2, a2)):
            mr[...] = jnp.full((BM, 1), jnp.inf, jnp.float32)
            ar[...] = jnp.zeros((BM, 1), jnp.int32)

    for k, (mr, ar) in enumerate(((m0, a0), (m1, a1), (m2, a2))):
        @pl.when(chunk == k)
        def _merge(mr=mr, ar=ar):
            better = bmin < mr[...]
            ar[...] = jnp.where(better, bidx, ar[...])
            mr[...] = jnp.where(better, bmin, mr[...])

    @pl.when(n == _N_BLOCKS - 1)
    def _emit():
        accv = _bf16r(m0[...])
        acci = a0[...]
        tv = m0[...]
        for mr, ar in ((m1, a1), (m2, a2)):
            mk = mr[...]
            win = mk < accv
            accv = jnp.where(win, _bf16r(mk), accv)
            acci = jnp.where(win, ar[...], acci)
            tv = jnp.where(win, mk, tv)
        idx_ref[...] = acci
        minv_ref[...] = tv


def _argmin_call(z_flat, cw_pad, zsq2d, csq2d):
    grid = (z_flat.shape[0] // BM, _N_BLOCKS)
    return pl.pallas_call(
        _argmin_body,
        grid=grid,
        in_specs=[
            pl.BlockSpec((BM, 1), lambda m, n: (m, 0)),
            pl.BlockSpec((1, BN), lambda m, n: (0, n)),
            pl.BlockSpec((BM, LATENT), lambda m, n: (m, 0)),
            pl.BlockSpec((BN, LATENT), lambda m, n: (n, 0)),
        ],
        out_specs=[
            pl.BlockSpec((BM, 1), lambda m, n: (m, 0)),
            pl.BlockSpec((BM, 1), lambda m, n: (m, 0)),
        ],
        out_shape=[
            jax.ShapeDtypeStruct((z_flat.shape[0], 1), jnp.int32),
            jax.ShapeDtypeStruct((z_flat.shape[0], 1), jnp.float32),
        ],
        scratch_shapes=[pltpu.VMEM((BM, 1), jnp.float32)] * 3
                      + [pltpu.VMEM((BM, 1), jnp.int32)] * 3,
    )(zsq2d, csq2d, z_flat, cw_pad)


def kernel(z, codebook_weight):
    B, C, H, W = z.shape
    z_permuted = jnp.transpose(z, (0, 2, 3, 1))
    z_flat = jnp.reshape(z_permuted, (-1, LATENT))
    n_vec = z_flat.shape[0]

    zsq = jnp.sum(z_flat ** 2, axis=1)
    csq = jnp.sum(codebook_weight ** 2, axis=1)

    cw_pad = jnp.zeros((N_PAD, LATENT), jnp.float32)
    csq_pad = jnp.full((N_PAD,), PAD_SENTINEL, jnp.float32)
    for k in range(3):
        lo, hi = k * CHUNK, min((k + 1) * CHUNK, NUM_CODES)
        cw_pad = lax.dynamic_update_slice(cw_pad, codebook_weight[lo:hi],
                                          (k * CHUNK_PAD, 0))
        csq_pad = lax.dynamic_update_slice(csq_pad, csq[lo:hi],
                                           (k * CHUNK_PAD,))

    idx2d, minv2d = _argmin_call(z_flat, cw_pad, zsq[:, None], csq_pad[None, :])
    indices = idx2d[:, 0]
    minv = minv2d[:, 0]

    z_q = jnp.take(codebook_weight, indices, axis=0)
    counts = jnp.zeros((NUM_CODES,), jnp.float32).at[indices].add(1.0)
    active = jnp.sum(counts > 0).astype(jnp.float32)
    utilization_pct = 100.0 * active / NUM_CODES
    avg_probs = counts / n_vec
    perplexity = jnp.exp(-jnp.sum(avg_probs * jnp.log(avg_probs + 1e-10)))
    embedding_loss = jnp.sum(minv) / (n_vec * LATENT)
    codebook_loss = BETA * embedding_loss

    z_q_out = jnp.transpose(z_q.reshape(B, H, W, C), (0, 3, 1, 2))
    return (z_q_out, indices, codebook_loss, perplexity,
            utilization_pct, embedding_loss)


# speedup vs baseline: 1.3494x; 1.3494x over previous
import functools

import jax
import jax.numpy as jnp
from jax import lax
from jax.experimental import pallas as pl
from jax.experimental.pallas import tpu as pltpu
from jax.experimental.pallas import tpu_sc as plsc

NUM_CODES = 8192
LATENT = 256
BETA = 0.25

CHUNK = 2736
CHUNK_PAD = 2816
N_PAD = 3 * CHUNK_PAD
PAD_SENTINEL = 1e30

BM = 512
BN = 1408
_N_BLOCKS = N_PAD // BN
_BIG_IDX = 2 ** 30


def _bf16r(x):
    return x.astype(jnp.bfloat16).astype(jnp.float32)


def _argmin_body(zsq_ref, csq_ref, z_ref, c_ref, idx_ref, minv_ref,
                 m0, m1, m2, a0, a1, a2):
    n = pl.program_id(1)
    zb = z_ref[...]
    cb = c_ref[...]
    mm = lax.dot_general(zb.astype(jnp.bfloat16), cb.astype(jnp.bfloat16),
                         (((1,), (1,)), ((), ())),
                         preferred_element_type=jnp.float32)
    dist = (zsq_ref[...] + csq_ref[...]) - mm * 2.0
    chunk = n // 2
    col0 = n * BN - chunk * (CHUNK_PAD - CHUNK)
    jidx = lax.broadcasted_iota(jnp.int32, (BM, BN), 1) + col0
    bmin = jnp.min(dist, axis=1, keepdims=True)
    bidx = jnp.min(jnp.where(dist == bmin, jidx, _BIG_IDX),
                   axis=1, keepdims=True)

    @pl.when(n == 0)
    def _init():
        for mr, ar in ((m0, a0), (m1, a1), (m2, a2)):
            mr[...] = jnp.full((BM, 1), jnp.inf, jnp.float32)
            ar[...] = jnp.zeros((BM, 1), jnp.int32)

    for k, (mr, ar) in enumerate(((m0, a0), (m1, a1), (m2, a2))):
        @pl.when(chunk == k)
        def _merge(mr=mr, ar=ar):
            better = bmin < mr[...]
            ar[...] = jnp.where(better, bidx, ar[...])
            mr[...] = jnp.where(better, bmin, mr[...])

    @pl.when(n == _N_BLOCKS - 1)
    def _emit():
        accv = _bf16r(m0[...])
        acci = a0[...]
        tv = m0[...]
        for mr, ar in ((m1, a1), (m2, a2)):
            mk = mr[...]
            win = mk < accv
            accv = jnp.where(win, _bf16r(mk), accv)
            acci = jnp.where(win, ar[...], acci)
            tv = jnp.where(win, mk, tv)
        idx_ref[...] = acci
        minv_ref[...] = tv


def _argmin_call(z_flat, cw_pad, zsq2d, csq2d):
    grid = (z_flat.shape[0] // BM, _N_BLOCKS)
    return pl.pallas_call(
        _argmin_body,
        grid=grid,
        in_specs=[
            pl.BlockSpec((BM, 1), lambda m, n: (m, 0)),
            pl.BlockSpec((1, BN), lambda m, n: (0, n)),
            pl.BlockSpec((BM, LATENT), lambda m, n: (m, 0)),
            pl.BlockSpec((BN, LATENT), lambda m, n: (n, 0)),
        ],
        out_specs=[
            pl.BlockSpec((BM, 1), lambda m, n: (m, 0)),
            pl.BlockSpec((BM, 1), lambda m, n: (m, 0)),
        ],
        out_shape=[
            jax.ShapeDtypeStruct((z_flat.shape[0], 1), jnp.int32),
            jax.ShapeDtypeStruct((z_flat.shape[0], 1), jnp.float32),
        ],
        scratch_shapes=[pltpu.VMEM((BM, 1), jnp.float32)] * 3
                      + [pltpu.VMEM((BM, 1), jnp.int32)] * 3,
    )(zsq2d, csq2d, z_flat, cw_pad)


_SC_MESH = plsc.VectorSubcoreMesh(core_axis_name="c", subcore_axis_name="s")


def _sc_body(idx_hbm, cw_hbm, ones_hbm, zeros_hbm, zq_hbm, cnt_hbm,
             idx_v, rows_v, ones_v, cnt_sh, sem):
    c = lax.axis_index("c")
    s = lax.axis_index("s")
    w = c * 16 + s
    r0 = w * 4
    pltpu.sync_copy(idx_hbm.at[pl.ds(r0, 4)], idx_v)
    pltpu.sync_copy(ones_hbm, ones_v)
    for j in range(4):
        pltpu.async_copy(cw_hbm.at[idx_v.at[j]], rows_v, sem).wait()
        pltpu.sync_copy(rows_v, zq_hbm.at[pl.ds(r0 * 128 + j * 128, 128)])

    @pl.when(s == 0)
    def _zero():
        pltpu.sync_copy(zeros_hbm, cnt_sh)

    plsc.subcore_barrier()
    for j in range(4):
        pltpu.sync_copy(ones_v, cnt_sh.at[idx_v.at[j]], add=True)
    plsc.subcore_barrier()
    pltpu.sync_copy(cnt_sh.at[pl.ds(s * 512, 512)],
                    cnt_hbm.at[c, pl.ds(s * 512, 512)])


@functools.partial(
    pl.kernel,
    out_type=[
        jax.ShapeDtypeStruct((16384, LATENT), jnp.float32),
        jax.ShapeDtypeStruct((2, NUM_CODES), jnp.float32),
    ],
    mesh=_SC_MESH,
    scratch_types=[
        pltpu.VMEM((4, 128), jnp.int32),
        pltpu.VMEM((128, LATENT), jnp.float32),
        pltpu.VMEM((128,), jnp.float32),
        pltpu.VMEM_SHARED((NUM_CODES,), jnp.float32),
        pltpu.SemaphoreType.DMA,
    ],
)
def _sc_gather_hist(idx_hbm, cw_hbm, ones_hbm, zeros_hbm, zq_hbm, cnt_hbm,
                    idx_v, rows_v, ones_v, cnt_sh, sem):
    _sc_body(idx_hbm, cw_hbm, ones_hbm, zeros_hbm, zq_hbm, cnt_hbm,
             idx_v, rows_v, ones_v, cnt_sh, sem)


def _stats_body(cnt_ref, minv_ref, cb_ref, ppl_ref, util_ref, emb_ref):
    cnt = cnt_ref[...]
    counts = cnt[0:1, :] + cnt[1:2, :]
    active = jnp.sum((counts > 0).astype(jnp.float32))
    avg = counts * (1.0 / 16384.0)
    ent = jnp.sum(avg * jnp.log(avg + 1e-10))
    emb = jnp.sum(minv_ref[...]) * (1.0 / (16384.0 * LATENT))
    cb_ref[...] = jnp.reshape(BETA * emb, (1, 1))
    ppl_ref[...] = jnp.reshape(jnp.exp(-ent), (1, 1))
    util_ref[...] = jnp.reshape(100.0 * active / NUM_CODES, (1, 1))
    emb_ref[...] = jnp.reshape(emb, (1, 1))


def _stats_call(cnt2, minv128):
    return pl.pallas_call(
        _stats_body,
        out_shape=[jax.ShapeDtypeStruct((1, 1), jnp.float32)] * 4,
    )(cnt2, minv128)


def kernel(z, codebook_weight):
    B, C, H, W = z.shape
    z_permuted = jnp.transpose(z, (0, 2, 3, 1))
    z_flat = jnp.reshape(z_permuted, (-1, LATENT))
    n_vec = z_flat.shape[0]

    zsq = jnp.sum(z_flat ** 2, axis=1)
    csq = jnp.sum(codebook_weight ** 2, axis=1)

    cw_pad = jnp.zeros((N_PAD, LATENT), jnp.float32)
    csq_pad = jnp.full((N_PAD,), PAD_SENTINEL, jnp.float32)
    for k in range(3):
        lo, hi = k * CHUNK, min((k + 1) * CHUNK, NUM_CODES)
        cw_pad = lax.dynamic_update_slice(cw_pad, codebook_weight[lo:hi],
                                          (k * CHUNK_PAD, 0))
        csq_pad = lax.dynamic_update_slice(csq_pad, csq[lo:hi],
                                           (k * CHUNK_PAD,))

    idx2d, minv2d = _argmin_call(z_flat, cw_pad, zsq[:, None], csq_pad[None, :])
    indices = idx2d[:, 0]

    z_q, cnt2 = _sc_gather_hist(
        jnp.reshape(indices, (128, 128)), codebook_weight,
        jnp.ones((128,), jnp.float32), jnp.zeros((NUM_CODES,), jnp.float32))

    cb, ppl, util, emb = _stats_call(cnt2, jnp.reshape(minv2d, (128, 128)))
    codebook_loss = jnp.reshape(cb, ())
    perplexity = jnp.reshape(ppl, ())
    utilization_pct = jnp.reshape(util, ())
    embedding_loss = jnp.reshape(emb, ())

    z_q_out = jnp.transpose(z_q.reshape(B, H, W, C), (0, 3, 1, 2))
    return (z_q_out, indices, codebook_loss, perplexity,
            utilization_pct, embedding_loss)


# speedup vs baseline: 1.6784x; 1.2438x over previous
import functools

import jax
import jax.numpy as jnp
from jax import lax
from jax.experimental import pallas as pl
from jax.experimental.pallas import tpu as pltpu
from jax.experimental.pallas import tpu_sc as plsc

NUM_CODES = 8192
LATENT = 256
BETA = 0.25

CHUNK = 2736
CHUNK_PAD = 2816
N_PAD = 3 * CHUNK_PAD
PAD_SENTINEL = 1e30

BM = 512
BN = 2816
_N_BLOCKS = N_PAD // BN
_BIG_IDX = 2 ** 30


def _bf16r(x):
    return x.astype(jnp.bfloat16).astype(jnp.float32)


def _argmin_body(zsq_ref, csq_ref, z_ref, c_ref, idx_ref, minv_ref,
                 m0, m1, m2, a0, a1, a2):
    n = pl.program_id(1)
    zb = z_ref[...]
    cb = c_ref[...]
    mm = lax.dot_general(zb.astype(jnp.bfloat16), cb.astype(jnp.bfloat16),
                         (((1,), (1,)), ((), ())),
                         preferred_element_type=jnp.float32)
    dist = (zsq_ref[...] + csq_ref[...]) - mm * 2.0
    chunk = n
    jidx = lax.broadcasted_iota(jnp.int32, (BM, BN), 1)
    bmin = jnp.min(dist, axis=1, keepdims=True)
    bidx = jnp.min(jnp.where(dist == bmin, jidx, _BIG_IDX),
                   axis=1, keepdims=True) + n * CHUNK

    @pl.when(n == 0)
    def _init():
        for mr, ar in ((m0, a0), (m1, a1), (m2, a2)):
            mr[...] = jnp.full((BM, 1), jnp.inf, jnp.float32)
            ar[...] = jnp.zeros((BM, 1), jnp.int32)

    for k, (mr, ar) in enumerate(((m0, a0), (m1, a1), (m2, a2))):
        @pl.when(chunk == k)
        def _merge(mr=mr, ar=ar):
            better = bmin < mr[...]
            ar[...] = jnp.where(better, bidx, ar[...])
            mr[...] = jnp.where(better, bmin, mr[...])

    @pl.when(n == _N_BLOCKS - 1)
    def _emit():
        accv = _bf16r(m0[...])
        acci = a0[...]
        tv = m0[...]
        for mr, ar in ((m1, a1), (m2, a2)):
            mk = mr[...]
            win = mk < accv
            accv = jnp.where(win, _bf16r(mk), accv)
            acci = jnp.where(win, ar[...], acci)
            tv = jnp.where(win, mk, tv)
        idx_ref[...] = acci
        minv_ref[...] = tv


def _argmin_call(z_flat, cw_pad, zsq2d, csq2d):
    grid = (z_flat.shape[0] // BM, _N_BLOCKS)
    return pl.pallas_call(
        _argmin_body,
        grid=grid,
        in_specs=[
            pl.BlockSpec((BM, 1), lambda m, n: (m, 0)),
            pl.BlockSpec((1, BN), lambda m, n: (0, n)),
            pl.BlockSpec((BM, LATENT), lambda m, n: (m, 0)),
            pl.BlockSpec((BN, LATENT), lambda m, n: (n, 0)),
        ],
        out_specs=[
            pl.BlockSpec((BM, 1), lambda m, n: (m, 0)),
            pl.BlockSpec((BM, 1), lambda m, n: (m, 0)),
        ],
        out_shape=[
            jax.ShapeDtypeStruct((z_flat.shape[0], 1), jnp.int32),
            jax.ShapeDtypeStruct((z_flat.shape[0], 1), jnp.float32),
        ],
        scratch_shapes=[pltpu.VMEM((BM, 1), jnp.float32)] * 3
                      + [pltpu.VMEM((BM, 1), jnp.int32)] * 3,
    )(zsq2d, csq2d, z_flat, cw_pad)


_SC_MESH = plsc.VectorSubcoreMesh(core_axis_name="c", subcore_axis_name="s")


def _sc_body(idx_hbm, cw_hbm, ones_hbm, zeros_hbm, zq_hbm, cnt_hbm,
             idx_v, rows_v, ones_v, cnt_sh, sem):
    c = lax.axis_index("c")
    s = lax.axis_index("s")
    w = c * 16 + s
    r0 = w * 4
    pltpu.sync_copy(idx_hbm.at[pl.ds(r0, 4)], idx_v)
    pltpu.sync_copy(ones_hbm, ones_v)
    for j in range(4):
        pltpu.async_copy(cw_hbm.at[idx_v.at[j]], rows_v, sem).wait()
        pltpu.sync_copy(rows_v, zq_hbm.at[pl.ds(r0 * 128 + j * 128, 128)])

    @pl.when(s == 0)
    def _zero():
        pltpu.sync_copy(zeros_hbm, cnt_sh)

    plsc.subcore_barrier()
    for j in range(4):
        pltpu.sync_copy(ones_v, cnt_sh.at[idx_v.at[j]], add=True)
    plsc.subcore_barrier()
    pltpu.sync_copy(cnt_sh.at[pl.ds(s * 512, 512)],
                    cnt_hbm.at[c, pl.ds(s * 512, 512)])


@functools.partial(
    pl.kernel,
    out_type=[
        jax.ShapeDtypeStruct((16384, LATENT), jnp.float32),
        jax.ShapeDtypeStruct((2, NUM_CODES), jnp.float32),
    ],
    mesh=_SC_MESH,
    scratch_types=[
        pltpu.VMEM((4, 128), jnp.int32),
        pltpu.VMEM((128, LATENT), jnp.float32),
        pltpu.VMEM((128,), jnp.float32),
        pltpu.VMEM_SHARED((NUM_CODES,), jnp.float32),
        pltpu.SemaphoreType.DMA,
    ],
)
def _sc_gather_hist(idx_hbm, cw_hbm, ones_hbm, zeros_hbm, zq_hbm, cnt_hbm,
                    idx_v, rows_v, ones_v, cnt_sh, sem):
    _sc_body(idx_hbm, cw_hbm, ones_hbm, zeros_hbm, zq_hbm, cnt_hbm,
             idx_v, rows_v, ones_v, cnt_sh, sem)


def _stats_body(cnt_ref, minv_ref, cb_ref, ppl_ref, util_ref, emb_ref):
    cnt = cnt_ref[...]
    counts = cnt[0:1, :] + cnt[1:2, :]
    active = jnp.sum((counts > 0).astype(jnp.float32))
    avg = counts * (1.0 / 16384.0)
    ent = jnp.sum(avg * jnp.log(avg + 1e-10))
    emb = jnp.sum(minv_ref[...]) * (1.0 / (16384.0 * LATENT))
    cb_ref[...] = jnp.reshape(BETA * emb, (1, 1))
    ppl_ref[...] = jnp.reshape(jnp.exp(-ent), (1, 1))
    util_ref[...] = jnp.reshape(100.0 * active / NUM_CODES, (1, 1))
    emb_ref[...] = jnp.reshape(emb, (1, 1))


def _stats_call(cnt2, minv128):
    return pl.pallas_call(
        _stats_body,
        out_shape=[jax.ShapeDtypeStruct((1, 1), jnp.float32)] * 4,
    )(cnt2, minv128)


def kernel(z, codebook_weight):
    B, C, H, W = z.shape
    z_permuted = jnp.transpose(z, (0, 2, 3, 1))
    z_flat = jnp.reshape(z_permuted, (-1, LATENT))
    n_vec = z_flat.shape[0]

    zsq = jnp.sum(z_flat ** 2, axis=1)
    csq = jnp.sum(codebook_weight ** 2, axis=1)

    cw_pad = jnp.zeros((N_PAD, LATENT), jnp.float32)
    csq_pad = jnp.full((N_PAD,), PAD_SENTINEL, jnp.float32)
    for k in range(3):
        lo, hi = k * CHUNK, min((k + 1) * CHUNK, NUM_CODES)
        cw_pad = lax.dynamic_update_slice(cw_pad, codebook_weight[lo:hi],
                                          (k * CHUNK_PAD, 0))
        csq_pad = lax.dynamic_update_slice(csq_pad, csq[lo:hi],
                                           (k * CHUNK_PAD,))

    idx2d, minv2d = _argmin_call(z_flat, cw_pad, zsq[:, None], csq_pad[None, :])
    indices = idx2d[:, 0]

    z_q, cnt2 = _sc_gather_hist(
        jnp.reshape(indices, (128, 128)), codebook_weight,
        jnp.ones((128,), jnp.float32), jnp.zeros((NUM_CODES,), jnp.float32))

    cb, ppl, util, emb = _stats_call(cnt2, jnp.reshape(minv2d, (128, 128)))
    codebook_loss = jnp.reshape(cb, ())
    perplexity = jnp.reshape(ppl, ())
    utilization_pct = jnp.reshape(util, ())
    embedding_loss = jnp.reshape(emb, ())

    z_q_out = jnp.transpose(z_q.reshape(B, H, W, C), (0, 3, 1, 2))
    return (z_q_out, indices, codebook_loss, perplexity,
            utilization_pct, embedding_loss)


# speedup vs baseline: 1.8064x; 1.0762x over previous
import functools

import jax
import jax.numpy as jnp
from jax import lax
from jax.experimental import pallas as pl
from jax.experimental.pallas import tpu as pltpu
from jax.experimental.pallas import tpu_sc as plsc

NUM_CODES = 8192
LATENT = 256
BETA = 0.25

CHUNK = 2736
CHUNK_PAD = 2816
N_PAD = 3 * CHUNK_PAD
PAD_SENTINEL = 1e30

BM = 1024
BN = 2816
_N_BLOCKS = N_PAD // BN
_BIG_IDX = 2 ** 30


def _bf16r(x):
    return x.astype(jnp.bfloat16).astype(jnp.float32)


def _argmin_body(zsq_ref, csq_ref, z_ref, c_ref, idx_ref, minv_ref,
                 m0, m1, m2, a0, a1, a2):
    n = pl.program_id(1)
    zb = z_ref[...]
    cb = c_ref[...]
    mm = lax.dot_general(zb.astype(jnp.bfloat16), cb.astype(jnp.bfloat16),
                         (((1,), (1,)), ((), ())),
                         preferred_element_type=jnp.float32)
    dist = (zsq_ref[...] + csq_ref[...]) - mm * 2.0
    chunk = n
    jidx = lax.broadcasted_iota(jnp.int32, (BM, BN), 1)
    bmin = jnp.min(dist, axis=1, keepdims=True)
    bidx = jnp.min(jnp.where(dist == bmin, jidx, _BIG_IDX),
                   axis=1, keepdims=True) + n * CHUNK

    @pl.when(n == 0)
    def _init():
        for mr, ar in ((m0, a0), (m1, a1), (m2, a2)):
            mr[...] = jnp.full((BM, 1), jnp.inf, jnp.float32)
            ar[...] = jnp.zeros((BM, 1), jnp.int32)

    for k, (mr, ar) in enumerate(((m0, a0), (m1, a1), (m2, a2))):
        @pl.when(chunk == k)
        def _merge(mr=mr, ar=ar):
            better = bmin < mr[...]
            ar[...] = jnp.where(better, bidx, ar[...])
            mr[...] = jnp.where(better, bmin, mr[...])

    @pl.when(n == _N_BLOCKS - 1)
    def _emit():
        accv = _bf16r(m0[...])
        acci = a0[...]
        tv = m0[...]
        for mr, ar in ((m1, a1), (m2, a2)):
            mk = mr[...]
            win = mk < accv
            accv = jnp.where(win, _bf16r(mk), accv)
            acci = jnp.where(win, ar[...], acci)
            tv = jnp.where(win, mk, tv)
        idx_ref[...] = acci
        minv_ref[...] = tv


def _argmin_call(z_flat, cw_pad, zsq2d, csq2d):
    grid = (z_flat.shape[0] // BM, _N_BLOCKS)
    return pl.pallas_call(
        _argmin_body,
        grid=grid,
        in_specs=[
            pl.BlockSpec((BM, 1), lambda m, n: (m, 0)),
            pl.BlockSpec((1, BN), lambda m, n: (0, n)),
            pl.BlockSpec((BM, LATENT), lambda m, n: (m, 0)),
            pl.BlockSpec((BN, LATENT), lambda m, n: (n, 0)),
        ],
        out_specs=[
            pl.BlockSpec((BM, 1), lambda m, n: (m, 0)),
            pl.BlockSpec((BM, 1), lambda m, n: (m, 0)),
        ],
        out_shape=[
            jax.ShapeDtypeStruct((z_flat.shape[0], 1), jnp.int32),
            jax.ShapeDtypeStruct((z_flat.shape[0], 1), jnp.float32),
        ],
        scratch_shapes=[pltpu.VMEM((BM, 1), jnp.float32)] * 3
                      + [pltpu.VMEM((BM, 1), jnp.int32)] * 3,
    )(zsq2d, csq2d, z_flat, cw_pad)


_SC_MESH = plsc.VectorSubcoreMesh(core_axis_name="c", subcore_axis_name="s")


def _sc_body(idx_hbm, cw_hbm, ones_hbm, zeros_hbm, zq_hbm, cnt_hbm,
             idx_v, rows_v, ones_v, cnt_sh, sem):
    c = lax.axis_index("c")
    s = lax.axis_index("s")
    w = c * 16 + s
    r0 = w * 4
    pltpu.sync_copy(idx_hbm.at[pl.ds(r0, 4)], idx_v)
    pltpu.sync_copy(ones_hbm, ones_v)
    for j in range(4):
        pltpu.async_copy(cw_hbm.at[idx_v.at[j]], rows_v, sem).wait()
        pltpu.sync_copy(rows_v, zq_hbm.at[pl.ds(r0 * 128 + j * 128, 128)])

    @pl.when(s == 0)
    def _zero():
        pltpu.sync_copy(zeros_hbm, cnt_sh)

    plsc.subcore_barrier()
    for j in range(4):
        pltpu.sync_copy(ones_v, cnt_sh.at[idx_v.at[j]], add=True)
    plsc.subcore_barrier()
    pltpu.sync_copy(cnt_sh.at[pl.ds(s * 512, 512)],
                    cnt_hbm.at[c, pl.ds(s * 512, 512)])


@functools.partial(
    pl.kernel,
    out_type=[
        jax.ShapeDtypeStruct((16384, LATENT), jnp.float32),
        jax.ShapeDtypeStruct((2, NUM_CODES), jnp.float32),
    ],
    mesh=_SC_MESH,
    scratch_types=[
        pltpu.VMEM((4, 128), jnp.int32),
        pltpu.VMEM((128, LATENT), jnp.float32),
        pltpu.VMEM((128,), jnp.float32),
        pltpu.VMEM_SHARED((NUM_CODES,), jnp.float32),
        pltpu.SemaphoreType.DMA,
    ],
)
def _sc_gather_hist(idx_hbm, cw_hbm, ones_hbm, zeros_hbm, zq_hbm, cnt_hbm,
                    idx_v, rows_v, ones_v, cnt_sh, sem):
    _sc_body(idx_hbm, cw_hbm, ones_hbm, zeros_hbm, zq_hbm, cnt_hbm,
             idx_v, rows_v, ones_v, cnt_sh, sem)


def _stats_body(cnt_ref, minv_ref, cb_ref, ppl_ref, util_ref, emb_ref):
    cnt = cnt_ref[...]
    counts = cnt[0:1, :] + cnt[1:2, :]
    active = jnp.sum((counts > 0).astype(jnp.float32))
    avg = counts * (1.0 / 16384.0)
    ent = jnp.sum(avg * jnp.log(avg + 1e-10))
    emb = jnp.sum(minv_ref[...]) * (1.0 / (16384.0 * LATENT))
    cb_ref[...] = jnp.reshape(BETA * emb, (1, 1))
    ppl_ref[...] = jnp.reshape(jnp.exp(-ent), (1, 1))
    util_ref[...] = jnp.reshape(100.0 * active / NUM_CODES, (1, 1))
    emb_ref[...] = jnp.reshape(emb, (1, 1))


def _stats_call(cnt2, minv128):
    return pl.pallas_call(
        _stats_body,
        out_shape=[jax.ShapeDtypeStruct((1, 1), jnp.float32)] * 4,
    )(cnt2, minv128)


def kernel(z, codebook_weight):
    B, C, H, W = z.shape
    z_permuted = jnp.transpose(z, (0, 2, 3, 1))
    z_flat = jnp.reshape(z_permuted, (-1, LATENT))
    n_vec = z_flat.shape[0]

    zsq = jnp.sum(z_flat ** 2, axis=1)
    csq = jnp.sum(codebook_weight ** 2, axis=1)

    cw_pad = jnp.zeros((N_PAD, LATENT), jnp.float32)
    csq_pad = jnp.full((N_PAD,), PAD_SENTINEL, jnp.float32)
    for k in range(3):
        lo, hi = k * CHUNK, min((k + 1) * CHUNK, NUM_CODES)
        cw_pad = lax.dynamic_update_slice(cw_pad, codebook_weight[lo:hi],
                                          (k * CHUNK_PAD, 0))
        csq_pad = lax.dynamic_update_slice(csq_pad, csq[lo:hi],
                                           (k * CHUNK_PAD,))

    idx2d, minv2d = _argmin_call(z_flat, cw_pad, zsq[:, None], csq_pad[None, :])
    indices = idx2d[:, 0]

    z_q, cnt2 = _sc_gather_hist(
        jnp.reshape(indices, (128, 128)), codebook_weight,
        jnp.ones((128,), jnp.float32), jnp.zeros((NUM_CODES,), jnp.float32))

    cb, ppl, util, emb = _stats_call(cnt2, jnp.reshape(minv2d, (128, 128)))
    codebook_loss = jnp.reshape(cb, ())
    perplexity = jnp.reshape(ppl, ())
    utilization_pct = jnp.reshape(util, ())
    embedding_loss = jnp.reshape(emb, ())

    z_q_out = jnp.transpose(z_q.reshape(B, H, W, C), (0, 3, 1, 2))
    return (z_q_out, indices, codebook_loss, perplexity,
            utilization_pct, embedding_loss)
